# baseline (speedup 1.0000x reference)
import functools

import jax
import jax.numpy as jnp
from jax import lax
from jax.experimental import pallas as pl
from jax.experimental.pallas import tpu as pltpu
from jax.experimental.pallas import tpu_sc as plsc

_B, _S, _D, _H, _HEADS, _VOCAB = 64, 128, 512, 1024, 8, 30000
_DH = _H // _HEADS
_BB = 2

_NC, _NS = 2, 16
_NW = _NC * _NS
_ROWS = _B * _S
_RPW = _ROWS // _NW
_CHUNK = 128


def _emb_gather(idx, table):
  mesh = plsc.VectorSubcoreMesh(core_axis_name="c", subcore_axis_name="s")

  @functools.partial(
      pl.kernel, mesh=mesh,
      out_type=jax.ShapeDtypeStruct((_ROWS, _D), jnp.float32),
      scratch_types=[
          pltpu.VMEM((_CHUNK,), jnp.int32),
          pltpu.VMEM((_CHUNK, _D), jnp.float32),
          pltpu.SemaphoreType.DMA,
      ],
  )
  def gather_kernel(idx_hbm, table_hbm, out_hbm, idx_v, rows_v, sem):
    wid = lax.axis_index("s") * _NC + lax.axis_index("c")
    base = wid * _RPW
    for c in range(_RPW // _CHUNK):
      off = base + c * _CHUNK
      pltpu.sync_copy(idx_hbm.at[pl.ds(off, _CHUNK)], idx_v)
      pltpu.async_copy(table_hbm.at[idx_v], rows_v, sem).wait()
      pltpu.sync_copy(rows_v, out_hbm.at[pl.ds(off, _CHUNK)])

  return gather_kernel(idx, table)


def _lrelu(x):
  return jnp.where(x >= 0.0, x, 0.2 * x)


def _elu(x):
  return jnp.where(x > 0.0, x, jnp.exp(jnp.where(x > 0.0, 0.0, x)) - 1.0)


def _masked_softmax(e, mask):
  e = jnp.where(mask, e, -1e9)
  e = e - jnp.max(e, axis=-1, keepdims=True)
  p = jnp.exp(e)
  return p / jnp.sum(p, axis=-1, keepdims=True)


_BF = jnp.bfloat16


def _tc_body(x_ref, a1_ref, a2_ref, a3_ref, w1cat_ref, sb1_ref, sw2_ref,
             sb2_ref, cb1_ref, cw2_ref, cb2_ref, gw2_ref, asrct_ref, adst_ref,
             a2src_ref, a2dst_ref, outw_ref, outb_ref, o_ref):
  xx = x_ref[...].reshape(_BB * _S, _D)
  t_all = jnp.dot(xx, w1cat_ref[...], preferred_element_type=jnp.float32)

  hs_l, hc_l, h1_l, d1_l, d2_l, a1b_l, a2b_l = [], [], [], [], [], [], []
  for i in range(_BB):
    t = t_all[i * _S:(i + 1) * _S]
    a1 = a1_ref[i]
    a2 = a2_ref[i]
    a3 = a3_ref[i]
    a1b = a1.astype(_BF)
    a2b = a2.astype(_BF)
    d1 = jnp.sum(a1, axis=-1, keepdims=True) + 1.0
    d2 = jnp.sum(a2, axis=-1, keepdims=True) + 1.0
    hs = jnp.maximum(
        jnp.dot(a1b, t[:, :_H].astype(_BF),
                preferred_element_type=jnp.float32) / d1
        + sb1_ref[...], 0.0)
    hc = jnp.maximum(
        jnp.dot(a2b, t[:, _H:2 * _H].astype(_BF),
                preferred_element_type=jnp.float32) / d2
        + cb1_ref[...], 0.0)
    hg = t[:, 2 * _H:]
    m3 = a3 > 0.0
    parts = []
    for hd in range(_HEADS):
      hh = hg[:, hd * _DH:(hd + 1) * _DH]
      es = jnp.dot(hh, asrct_ref[...][:, hd:hd + 1],
                   preferred_element_type=jnp.float32)
      ed = lax.dot_general(adst_ref[hd:hd + 1, :], hh,
                           (((1,), (1,)), ((), ())),
                           preferred_element_type=jnp.float32)
      att = _masked_softmax(_lrelu(es + ed), m3)
      parts.append(jnp.dot(att.astype(_BF), hh.astype(_BF),
                           preferred_element_type=jnp.float32))
    h1 = _elu(jnp.concatenate(parts, axis=1))
    hs_l.append(hs.astype(_BF))
    hc_l.append(hc.astype(_BF))
    h1_l.append(h1.astype(_BF))
    d1_l.append(d1)
    d2_l.append(d2)
    a1b_l.append(a1b)
    a2b_l.append(a2b)

  t2s = jnp.dot(jnp.concatenate(hs_l, axis=0), sw2_ref[...],
                preferred_element_type=jnp.float32)
  t2c = jnp.dot(jnp.concatenate(hc_l, axis=0), cw2_ref[...],
                preferred_element_type=jnp.float32)
  h2_all = jnp.dot(jnp.concatenate(h1_l, axis=0), gw2_ref[...],
                   preferred_element_type=jnp.float32)

  for i in range(_BB):
    a3 = a3_ref[i]
    syn = jnp.maximum(
        jnp.dot(a1b_l[i], t2s[i * _S:(i + 1) * _S].astype(_BF),
                preferred_element_type=jnp.float32) / d1_l[i]
        + sb2_ref[...], 0.0)
    com = jnp.maximum(
        jnp.dot(a2b_l[i], t2c[i * _S:(i + 1) * _S].astype(_BF),
                preferred_element_type=jnp.float32) / d2_l[i]
        + cb2_ref[...], 0.0)
    h2 = h2_all[i * _S:(i + 1) * _S]
    es2 = jnp.dot(h2, a2src_ref[...], preferred_element_type=jnp.float32)
    ed2 = lax.dot_general(a2dst_ref[...], h2, (((1,), (1,)), ((), ())),
                          preferred_element_type=jnp.float32)
    att2 = _masked_softmax(_lrelu(es2 + ed2), a3 > 0.0)
    sem = _elu(jnp.dot(att2.astype(_BF), h2.astype(_BF),
                       preferred_element_type=jnp.float32))
    g = jnp.concatenate([syn, com, sem], axis=1)
    o_ref[i] = (jnp.dot(g, outw_ref[...], preferred_element_type=jnp.float32)
                + outb_ref[...])


def _tc_specs():
  def blk(b):
    return (b, 0, 0)

  def whole(b):
    return (0, 0)

  in_specs = [
      pl.BlockSpec((_BB, _S, _D), blk),
      pl.BlockSpec((_BB, _S, _S), blk),
      pl.BlockSpec((_BB, _S, _S), blk),
      pl.BlockSpec((_BB, _S, _S), blk),
      pl.BlockSpec((_D, 3 * _H), whole),
      pl.BlockSpec((1, _H), whole),
      pl.BlockSpec((_H, _D), whole),
      pl.BlockSpec((1, _D), whole),
      pl.BlockSpec((1, _H), whole),
      pl.BlockSpec((_H, _D), whole),
      pl.BlockSpec((1, _D), whole),
      pl.BlockSpec((_H, _D), whole),
      pl.BlockSpec((_DH, _HEADS), whole),
      pl.BlockSpec((_HEADS, _DH), whole),
      pl.BlockSpec((_D, 1), whole),
      pl.BlockSpec((1, _D), whole),
      pl.BlockSpec((3 * _D, 3), whole),
      pl.BlockSpec((1, 3), whole),
  ]
  return dict(
      grid=(_B // _BB,),
      in_specs=in_specs,
      out_specs=pl.BlockSpec((_BB, _S, 3), blk),
      out_shape=jax.ShapeDtypeStruct((_B, _S, 3), jnp.float32),
  )


def kernel(inputs, adj1, adj2, adj3, emb_table, syn_W1, syn_b1, syn_W2,
           syn_b2, com_W1, com_b1, com_W2, com_b2, gat_W1, gat_a_src,
           gat_a_dst, gat_W2, gat_a2_src, gat_a2_dst, out_W, out_b):
  idx = inputs.reshape(-1).astype(jnp.int32)
  x = _emb_gather(idx, emb_table).reshape(_B, _S, _D)
  w1g = jnp.transpose(gat_W1, (1, 0, 2)).reshape(_D, _H)
  w1cat = jnp.concatenate([syn_W1, com_W1, w1g], axis=1)
  return pl.pallas_call(
      _tc_body,
      compiler_params=pltpu.CompilerParams(
          dimension_semantics=("arbitrary",)),
      **_tc_specs(),
  )(x.astype(_BF), adj1, adj2, adj3, w1cat.astype(_BF),
    syn_b1.reshape(1, _H), syn_W2.astype(_BF), syn_b2.reshape(1, _D),
    com_b1.reshape(1, _H), com_W2.astype(_BF), com_b2.reshape(1, _D),
    gat_W2.astype(_BF), gat_a_src.T, gat_a_dst,
    gat_a2_src.reshape(_D, 1), gat_a2_dst.reshape(1, _D),
    out_W, out_b.reshape(1, 3))


# speedup vs baseline: 1.6227x; 1.6227x over previous
import functools

import jax
import jax.numpy as jnp
from jax import lax
from jax.experimental import pallas as pl
from jax.experimental.pallas import tpu as pltpu
from jax.experimental.pallas import tpu_sc as plsc

_B, _S, _D, _H, _HEADS, _VOCAB = 64, 128, 512, 1024, 8, 30000
_DH = _H // _HEADS
_BB = 2

_NC, _NS = 2, 16
_NW = _NC * _NS
_ROWS = _B * _S
_RPW = _ROWS // _NW
_CHUNK = 128


def _emb_gather(idx, table):
  mesh = plsc.VectorSubcoreMesh(core_axis_name="c", subcore_axis_name="s")

  @functools.partial(
      pl.kernel, mesh=mesh,
      out_type=jax.ShapeDtypeStruct((_ROWS, _D), jnp.float32),
      scratch_types=[
          pltpu.VMEM((_CHUNK,), jnp.int32),
          pltpu.VMEM((_CHUNK, _D), jnp.float32),
          pltpu.SemaphoreType.DMA,
      ],
  )
  def gather_kernel(idx_hbm, table_hbm, out_hbm, idx_v, rows_v, sem):
    wid = lax.axis_index("s") * _NC + lax.axis_index("c")
    base = wid * _RPW
    for c in range(_RPW // _CHUNK):
      off = base + c * _CHUNK
      pltpu.sync_copy(idx_hbm.at[pl.ds(off, _CHUNK)], idx_v)
      pltpu.async_copy(table_hbm.at[idx_v], rows_v, sem).wait()
      pltpu.sync_copy(rows_v, out_hbm.at[pl.ds(off, _CHUNK)])

  return gather_kernel(idx, table)


def _lrelu(x):
  return jnp.where(x >= 0.0, x, 0.2 * x)


def _elu(x):
  return jnp.where(x > 0.0, x, jnp.exp(jnp.where(x > 0.0, 0.0, x)) - 1.0)


def _masked_softmax(e, mask):
  e = jnp.where(mask, e, -1e9)
  e = e - jnp.max(e, axis=-1, keepdims=True)
  p = jnp.exp(e)
  return p / jnp.sum(p, axis=-1, keepdims=True)


_BF = jnp.bfloat16


def _tc_body(x_ref, a1_ref, a2_ref, a3_ref, w1cat_ref, sb1_ref, sw2_ref,
             sb2_ref, cb1_ref, cw2_ref, cb2_ref, gw2_ref, asrct_ref, adst_ref,
             a2src_ref, a2dst_ref, outw_ref, outb_ref, o_ref):
  xx = x_ref[...].reshape(_BB * _S, _D)
  t_all = jnp.dot(xx, w1cat_ref[...], preferred_element_type=jnp.float32)

  hs_l, hc_l, h1_l, d1_l, d2_l, a1b_l, a2b_l = [], [], [], [], [], [], []
  for i in range(_BB):
    t = t_all[i * _S:(i + 1) * _S]
    a1 = a1_ref[i]
    a2 = a2_ref[i]
    a3 = a3_ref[i]
    a1b = a1.astype(_BF)
    a2b = a2.astype(_BF)
    d1 = jnp.sum(a1, axis=-1, keepdims=True) + 1.0
    d2 = jnp.sum(a2, axis=-1, keepdims=True) + 1.0
    hs = jnp.maximum(
        jnp.dot(a1b, t[:, :_H].astype(_BF),
                preferred_element_type=jnp.float32) / d1
        + sb1_ref[...], 0.0)
    hc = jnp.maximum(
        jnp.dot(a2b, t[:, _H:2 * _H].astype(_BF),
                preferred_element_type=jnp.float32) / d2
        + cb1_ref[...], 0.0)
    hg = t[:, 2 * _H:]
    m3 = a3 > 0.0
    es_all = jnp.dot(hg, asrct_ref[...],
                     preferred_element_type=jnp.float32)
    edt_all = lax.dot_general(adst_ref[...], hg,
                              (((1,), (1,)), ((), ())),
                              preferred_element_type=jnp.float32)
    parts = []
    for hd in range(_HEADS):
      hh = hg[:, hd * _DH:(hd + 1) * _DH]
      es = es_all[:, hd:hd + 1]
      ed = edt_all[hd:hd + 1, :]
      att = _masked_softmax(_lrelu(es + ed), m3)
      parts.append(jnp.dot(att.astype(_BF), hh.astype(_BF),
                           preferred_element_type=jnp.float32))
    h1 = _elu(jnp.concatenate(parts, axis=1))
    hs_l.append(hs.astype(_BF))
    hc_l.append(hc.astype(_BF))
    h1_l.append(h1.astype(_BF))
    d1_l.append(d1)
    d2_l.append(d2)
    a1b_l.append(a1b)
    a2b_l.append(a2b)

  t2s = jnp.dot(jnp.concatenate(hs_l, axis=0), sw2_ref[...],
                preferred_element_type=jnp.float32)
  t2c = jnp.dot(jnp.concatenate(hc_l, axis=0), cw2_ref[...],
                preferred_element_type=jnp.float32)
  h2_all = jnp.dot(jnp.concatenate(h1_l, axis=0), gw2_ref[...],
                   preferred_element_type=jnp.float32)

  for i in range(_BB):
    a3 = a3_ref[i]
    syn = jnp.maximum(
        jnp.dot(a1b_l[i], t2s[i * _S:(i + 1) * _S].astype(_BF),
                preferred_element_type=jnp.float32) / d1_l[i]
        + sb2_ref[...], 0.0)
    com = jnp.maximum(
        jnp.dot(a2b_l[i], t2c[i * _S:(i + 1) * _S].astype(_BF),
                preferred_element_type=jnp.float32) / d2_l[i]
        + cb2_ref[...], 0.0)
    h2 = h2_all[i * _S:(i + 1) * _S]
    es2 = jnp.dot(h2, a2src_ref[...], preferred_element_type=jnp.float32)
    ed2 = lax.dot_general(a2dst_ref[...], h2, (((1,), (1,)), ((), ())),
                          preferred_element_type=jnp.float32)
    att2 = _masked_softmax(_lrelu(es2 + ed2), a3 > 0.0)
    sem = _elu(jnp.dot(att2.astype(_BF), h2.astype(_BF),
                       preferred_element_type=jnp.float32))
    g = jnp.concatenate([syn, com, sem], axis=1)
    o_ref[i] = (jnp.dot(g, outw_ref[...], preferred_element_type=jnp.float32)
                + outb_ref[...])


def _tc_specs():
  def blk(b):
    return (b, 0, 0)

  def whole(b):
    return (0, 0)

  in_specs = [
      pl.BlockSpec((_BB, _S, _D), blk),
      pl.BlockSpec((_BB, _S, _S), blk),
      pl.BlockSpec((_BB, _S, _S), blk),
      pl.BlockSpec((_BB, _S, _S), blk),
      pl.BlockSpec((_D, 3 * _H), whole),
      pl.BlockSpec((1, _H), whole),
      pl.BlockSpec((_H, _D), whole),
      pl.BlockSpec((1, _D), whole),
      pl.BlockSpec((1, _H), whole),
      pl.BlockSpec((_H, _D), whole),
      pl.BlockSpec((1, _D), whole),
      pl.BlockSpec((_H, _D), whole),
      pl.BlockSpec((_H, _HEADS), whole),
      pl.BlockSpec((_HEADS, _H), whole),
      pl.BlockSpec((_D, 1), whole),
      pl.BlockSpec((1, _D), whole),
      pl.BlockSpec((3 * _D, 3), whole),
      pl.BlockSpec((1, 3), whole),
  ]
  return dict(
      grid=(_B // _BB,),
      in_specs=in_specs,
      out_specs=pl.BlockSpec((_BB, _S, 3), blk),
      out_shape=jax.ShapeDtypeStruct((_B, _S, 3), jnp.float32),
  )


def kernel(inputs, adj1, adj2, adj3, emb_table, syn_W1, syn_b1, syn_W2,
           syn_b2, com_W1, com_b1, com_W2, com_b2, gat_W1, gat_a_src,
           gat_a_dst, gat_W2, gat_a2_src, gat_a2_dst, out_W, out_b):
  idx = inputs.reshape(-1).astype(jnp.int32)
  x = _emb_gather(idx, emb_table).reshape(_B, _S, _D)
  w1g = jnp.transpose(gat_W1, (1, 0, 2)).reshape(_D, _H)
  w1cat = jnp.concatenate([syn_W1, com_W1, w1g], axis=1)
  head_eye = jnp.eye(_HEADS, dtype=jnp.float32)
  asrc_mat = jnp.einsum('he,hk->hek', gat_a_src, head_eye).reshape(_H, _HEADS)
  adst_big = jnp.einsum('he,hk->hke', gat_a_dst, head_eye).reshape(_HEADS, _H)
  return pl.pallas_call(
      _tc_body,
      compiler_params=pltpu.CompilerParams(
          dimension_semantics=("arbitrary",)),
      **_tc_specs(),
  )(x.astype(_BF), adj1, adj2, adj3, w1cat.astype(_BF),
    syn_b1.reshape(1, _H), syn_W2.astype(_BF), syn_b2.reshape(1, _D),
    com_b1.reshape(1, _H), com_W2.astype(_BF), com_b2.reshape(1, _D),
    gat_W2.astype(_BF), asrc_mat, adst_big,
    gat_a2_src.reshape(_D, 1), gat_a2_dst.reshape(1, _D),
    out_W, out_b.reshape(1, 3))


# speedup vs baseline: 1.8794x; 1.1582x over previous
import functools

import jax
import jax.numpy as jnp
from jax import lax
from jax.experimental import pallas as pl
from jax.experimental.pallas import tpu as pltpu
from jax.experimental.pallas import tpu_sc as plsc

_B, _S, _D, _H, _HEADS, _VOCAB = 64, 128, 512, 1024, 8, 30000
_DH = _H // _HEADS
_BB = 4

_NC, _NS = 2, 16
_NW = _NC * _NS
_ROWS = _B * _S
_RPW = _ROWS // _NW
_CHUNK = 128


def _emb_gather(idx, table):
  mesh = plsc.VectorSubcoreMesh(core_axis_name="c", subcore_axis_name="s")

  @functools.partial(
      pl.kernel, mesh=mesh,
      out_type=jax.ShapeDtypeStruct((_ROWS, _D), jnp.float32),
      scratch_types=[
          pltpu.VMEM((_CHUNK,), jnp.int32),
          pltpu.VMEM((_CHUNK, _D), jnp.float32),
          pltpu.SemaphoreType.DMA,
      ],
  )
  def gather_kernel(idx_hbm, table_hbm, out_hbm, idx_v, rows_v, sem):
    wid = lax.axis_index("s") * _NC + lax.axis_index("c")
    base = wid * _RPW
    for c in range(_RPW // _CHUNK):
      off = base + c * _CHUNK
      pltpu.sync_copy(idx_hbm.at[pl.ds(off, _CHUNK)], idx_v)
      pltpu.async_copy(table_hbm.at[idx_v], rows_v, sem).wait()
      pltpu.sync_copy(rows_v, out_hbm.at[pl.ds(off, _CHUNK)])

  return gather_kernel(idx, table)


def _lrelu(x):
  return jnp.where(x >= 0.0, x, 0.2 * x)


def _elu(x):
  return jnp.where(x > 0.0, x, jnp.exp(jnp.where(x > 0.0, 0.0, x)) - 1.0)


def _masked_softmax(e, mask):
  e = jnp.where(mask, e, -1e9)
  e = e - jnp.max(e, axis=-1, keepdims=True)
  p = jnp.exp(e)
  return p / jnp.sum(p, axis=-1, keepdims=True)


_BF = jnp.bfloat16


def _tc_body(x_ref, a1_ref, a2_ref, a3_ref, w1cat_ref, sb1_ref, sw2_ref,
             sb2_ref, cb1_ref, cw2_ref, cb2_ref, gw2_ref, asrct_ref, adst_ref,
             a2src_ref, a2dst_ref, outw_ref, outb_ref, o_ref):
  xx = x_ref[...].reshape(_BB * _S, _D).astype(_BF)
  t_all = jnp.dot(xx, w1cat_ref[...], preferred_element_type=jnp.float32)

  hs_l, hc_l, h1_l, d1_l, d2_l, a1b_l, a2b_l = [], [], [], [], [], [], []
  for i in range(_BB):
    t = t_all[i * _S:(i + 1) * _S]
    a1 = a1_ref[i]
    a2 = a2_ref[i]
    a3 = a3_ref[i]
    a1b = a1.astype(_BF)
    a2b = a2.astype(_BF)
    d1 = jnp.sum(a1, axis=-1, keepdims=True) + 1.0
    d2 = jnp.sum(a2, axis=-1, keepdims=True) + 1.0
    hs = jnp.maximum(
        jnp.dot(a1b, t[:, :_H].astype(_BF),
                preferred_element_type=jnp.float32) / d1
        + sb1_ref[...], 0.0)
    hc = jnp.maximum(
        jnp.dot(a2b, t[:, _H:2 * _H].astype(_BF),
                preferred_element_type=jnp.float32) / d2
        + cb1_ref[...], 0.0)
    hg = t[:, 2 * _H:]
    m3 = a3 > 0.0
    es_all = jnp.dot(hg, asrct_ref[...],
                     preferred_element_type=jnp.float32)
    edt_all = lax.dot_general(adst_ref[...], hg,
                              (((1,), (1,)), ((), ())),
                              preferred_element_type=jnp.float32)
    parts = []
    for hd in range(_HEADS):
      hh = hg[:, hd * _DH:(hd + 1) * _DH]
      es = es_all[:, hd:hd + 1]
      ed = edt_all[hd:hd + 1, :]
      att = _masked_softmax(_lrelu(es + ed), m3)
      parts.append(jnp.dot(att.astype(_BF), hh.astype(_BF),
                           preferred_element_type=jnp.float32))
    h1 = _elu(jnp.concatenate(parts, axis=1))
    hs_l.append(hs.astype(_BF))
    hc_l.append(hc.astype(_BF))
    h1_l.append(h1.astype(_BF))
    d1_l.append(d1)
    d2_l.append(d2)
    a1b_l.append(a1b)
    a2b_l.append(a2b)

  t2s = jnp.dot(jnp.concatenate(hs_l, axis=0), sw2_ref[...],
                preferred_element_type=jnp.float32)
  t2c = jnp.dot(jnp.concatenate(hc_l, axis=0), cw2_ref[...],
                preferred_element_type=jnp.float32)
  h2_all = jnp.dot(jnp.concatenate(h1_l, axis=0), gw2_ref[...],
                   preferred_element_type=jnp.float32)
  es2_all = jnp.dot(h2_all, a2src_ref[...],
                    preferred_element_type=jnp.float32)
  ed2t_all = lax.dot_general(a2dst_ref[...], h2_all,
                             (((1,), (1,)), ((), ())),
                             preferred_element_type=jnp.float32)

  for i in range(_BB):
    a3 = a3_ref[i]
    syn = jnp.maximum(
        jnp.dot(a1b_l[i], t2s[i * _S:(i + 1) * _S].astype(_BF),
                preferred_element_type=jnp.float32) / d1_l[i]
        + sb2_ref[...], 0.0)
    com = jnp.maximum(
        jnp.dot(a2b_l[i], t2c[i * _S:(i + 1) * _S].astype(_BF),
                preferred_element_type=jnp.float32) / d2_l[i]
        + cb2_ref[...], 0.0)
    h2 = h2_all[i * _S:(i + 1) * _S]
    es2 = es2_all[i * _S:(i + 1) * _S]
    ed2 = ed2t_all[:, i * _S:(i + 1) * _S]
    att2 = _masked_softmax(_lrelu(es2 + ed2), a3 > 0.0)
    sem = _elu(jnp.dot(att2.astype(_BF), h2.astype(_BF),
                       preferred_element_type=jnp.float32))
    g = jnp.concatenate([syn, com, sem], axis=1)
    o_ref[i] = (jnp.dot(g, outw_ref[...], preferred_element_type=jnp.float32)
                + outb_ref[...])


def _tc_specs():
  def blk(b):
    return (b, 0, 0)

  def whole(b):
    return (0, 0)

  in_specs = [
      pl.BlockSpec((_BB, _S, _D), blk),
      pl.BlockSpec((_BB, _S, _S), blk),
      pl.BlockSpec((_BB, _S, _S), blk),
      pl.BlockSpec((_BB, _S, _S), blk),
      pl.BlockSpec((_D, 3 * _H), whole),
      pl.BlockSpec((1, _H), whole),
      pl.BlockSpec((_H, _D), whole),
      pl.BlockSpec((1, _D), whole),
      pl.BlockSpec((1, _H), whole),
      pl.BlockSpec((_H, _D), whole),
      pl.BlockSpec((1, _D), whole),
      pl.BlockSpec((_H, _D), whole),
      pl.BlockSpec((_H, _HEADS), whole),
      pl.BlockSpec((_HEADS, _H), whole),
      pl.BlockSpec((_D, 1), whole),
      pl.BlockSpec((1, _D), whole),
      pl.BlockSpec((3 * _D, 3), whole),
      pl.BlockSpec((1, 3), whole),
  ]
  return dict(
      grid=(_B // _BB,),
      in_specs=in_specs,
      out_specs=pl.BlockSpec((_BB, _S, 3), blk),
      out_shape=jax.ShapeDtypeStruct((_B, _S, 3), jnp.float32),
  )


def kernel(inputs, adj1, adj2, adj3, emb_table, syn_W1, syn_b1, syn_W2,
           syn_b2, com_W1, com_b1, com_W2, com_b2, gat_W1, gat_a_src,
           gat_a_dst, gat_W2, gat_a2_src, gat_a2_dst, out_W, out_b):
  idx = inputs.reshape(-1).astype(jnp.int32)
  x = _emb_gather(idx, emb_table).reshape(_B, _S, _D)
  w1g = jnp.transpose(gat_W1, (1, 0, 2)).reshape(_D, _H)
  w1cat = jnp.concatenate([syn_W1, com_W1, w1g], axis=1)
  head_eye = jnp.eye(_HEADS, dtype=jnp.float32)
  asrc_mat = jnp.einsum('he,hk->hek', gat_a_src, head_eye).reshape(_H, _HEADS)
  adst_big = jnp.einsum('he,hk->hke', gat_a_dst, head_eye).reshape(_HEADS, _H)
  return pl.pallas_call(
      _tc_body,
      compiler_params=pltpu.CompilerParams(
          dimension_semantics=("arbitrary",)),
      **_tc_specs(),
  )(x, adj1, adj2, adj3, w1cat.astype(_BF),
    syn_b1.reshape(1, _H), syn_W2.astype(_BF), syn_b2.reshape(1, _D),
    com_b1.reshape(1, _H), com_W2.astype(_BF), com_b2.reshape(1, _D),
    gat_W2.astype(_BF), asrc_mat, adst_big,
    gat_a2_src.reshape(_D, 1), gat_a2_dst.reshape(1, _D),
    out_W, out_b.reshape(1, 3))
